# baseline (speedup 1.0000x reference)
import functools

import jax
import jax.numpy as jnp
from jax import lax
from jax.experimental import pallas as pl
from jax.experimental.pallas import tpu as pltpu
from jax.experimental.pallas import tpu_sc as plsc

BATCH = 16384
EMBED_DIM = 128
_CHUNK = 128


_NBUF = 4


def _gather_tec_body(nc, bpw, uidx, iidx, utab, itab, cat_out,
                     uidx_v, iidx_v, buf, *sems):
    wid = lax.axis_index("s") * nc + lax.axis_index("c")
    base = wid * bpw
    nck = bpw // _CHUNK
    pltpu.sync_copy(uidx.at[pl.ds(base, bpw)], uidx_v)
    pltpu.sync_copy(iidx.at[pl.ds(base, bpw)], iidx_v)
    gsems = sems[:_NBUF]
    ssems = sems[_NBUF:]
    tasks = ([(uidx_v, utab, 0, j) for j in range(nck)]
             + [(iidx_v, itab, EMBED_DIM, j) for j in range(nck)])
    gathers = [None] * _NBUF
    stores = [None] * _NBUF

    def drain(t):
        b = t % _NBUF
        _, _, col0, j = tasks[t]
        gathers[b].wait()
        stores[b] = pltpu.async_copy(
            buf.at[b],
            cat_out.at[pl.ds(base + j * _CHUNK, _CHUNK),
                       pl.ds(col0, EMBED_DIM)],
            ssems[b])

    for t, (iv, tab, col0, j) in enumerate(tasks):
        b = t % _NBUF
        if stores[b] is not None:
            stores[b].wait()
        gathers[b] = pltpu.async_copy(
            tab.at[iv.at[pl.ds(j * _CHUNK, _CHUNK)]], buf.at[b], gsems[b])
        if t >= _NBUF - 1:
            drain(t - _NBUF + 1)
    for t in range(len(tasks) - _NBUF + 1, len(tasks)):
        drain(t)
    for s in stores:
        if s is not None:
            s.wait()


def _sc_gather(user_indices, item_indices, user_emb, item_emb):
    info = plsc.get_sparse_core_info()
    nc, ns = info.num_cores, info.num_subcores
    nw = nc * ns
    bpw = BATCH // nw
    mesh = plsc.VectorSubcoreMesh(core_axis_name="c", subcore_axis_name="s")
    k = pl.kernel(
        functools.partial(_gather_tec_body, nc, bpw),
        mesh=mesh,
        out_type=jax.ShapeDtypeStruct((BATCH, 2 * EMBED_DIM), jnp.float32),
        scratch_types=[
            pltpu.VMEM((bpw,), jnp.int32),
            pltpu.VMEM((bpw,), jnp.int32),
            pltpu.VMEM((_NBUF, _CHUNK, EMBED_DIM), jnp.float32),
        ] + [pltpu.SemaphoreType.DMA] * (2 * _NBUF),
    )
    return k(user_indices, item_indices, user_emb, item_emb)


def _mlp_body(xin, w1, b1, w2, b2, w3, b3, wo, bo, out):
    x = jnp.dot(xin[...].astype(jnp.bfloat16), w1[...],
                preferred_element_type=jnp.float32)
    x = jnp.maximum(x + b1[...], 0.0).astype(jnp.bfloat16)
    x = jnp.maximum(jnp.dot(x, w2[...], preferred_element_type=jnp.float32) + b2[...], 0.0)
    x = jnp.maximum(jnp.dot(x, w3[...], preferred_element_type=jnp.float32) + b3[...], 0.0)
    y = lax.dot_general(wo[...], x, (((1,), (1,)), ((), ())),
                        preferred_element_type=jnp.float32)
    out[...] = y.reshape(out.shape) + bo[0, 0]


def _tc_mlp(xcat, w1_t, b1, w2_t, b2, w3_t, b3, wo, bo):
    blk = 4096
    grid = BATCH // blk
    full = lambda shape: pl.BlockSpec(shape, lambda i: (0, 0))
    return pl.pallas_call(
        _mlp_body,
        grid=(grid,),
        in_specs=[
            pl.BlockSpec((blk, 2 * EMBED_DIM), lambda i: (i, 0)),
            full(w1_t.shape),
            full(b1.shape),
            full(w2_t.shape),
            full(b2.shape),
            full(w3_t.shape),
            full(b3.shape),
            full(wo.shape),
            full(bo.shape),
        ],
        out_specs=pl.BlockSpec((blk // 128, 128), lambda i: (i, 0)),
        out_shape=jax.ShapeDtypeStruct((BATCH // 128, 128), jnp.float32),
    )(xcat, w1_t, b1, w2_t, b2, w3_t, b3, wo, bo).reshape(BATCH)


def kernel(user_indices, item_indices, user_emb, item_emb,
           W1, b1, W2, b2, W3, b3, Wo, bo):
    user_indices = user_indices.astype(jnp.int32)
    item_indices = item_indices.astype(jnp.int32)
    xcat = _sc_gather(user_indices, item_indices, user_emb, item_emb)
    return _tc_mlp(
        xcat,
        W1.T.astype(jnp.bfloat16), b1.reshape(1, -1),
        W2.T.astype(jnp.bfloat16), b2.reshape(1, -1),
        W3.T, b3.reshape(1, -1),
        Wo, bo.reshape(1, 1),
    )
